# Initial kernel scaffold; baseline (speedup 1.0000x reference)
import jax
import jax.numpy as jnp
from jax.experimental import pallas as pl


def kernel(x, batch, lamb, gamma):
    raise NotImplementedError("write your pallas kernel here")


# speedup vs baseline: 3.3291x; 3.3291x over previous
import functools

import jax
import jax.numpy as jnp
from jax import lax
from jax.experimental import pallas as pl
from jax.experimental.pallas import tpu as pltpu
from jax.experimental.pallas import tpu_sc as plsc

N = 320000
D = 128
S = 10000
NC = 2
NS = 16
L = 16

COLS = D // NC
RPT = N // NS
CH = 256
NCH = RPT // CH
TAIL = RPT - NCH * CH
SEG_PT = S // NS
SEG_CH = 125
NSEG_CH = SEG_PT // SEG_CH


def _body(x_hbm, b_hbm, lg_hbm, out_hbm,
          sums_sh, counts_sh,
          xbuf, gbuf, obuf, idxbuf, tailidx, onesbuf, lgbuf):
  c = lax.axis_index("c")
  s = lax.axis_index("s")
  col0 = c * COLS
  base = s * RPT

  pltpu.sync_copy(lg_hbm, lgbuf)
  lamb_v = lgbuf[pl.ds(0, L)]
  gamma_v = lgbuf[pl.ds(L, L)]

  one_v = jnp.full((L,), 1.0, dtype=jnp.float32)
  zero_v = jnp.zeros((L,), dtype=jnp.float32)

  def init_row(i, carry):
    onesbuf[i, pl.ds(0, L)] = one_v
    return carry
  lax.fori_loop(0, 128, init_row, 0)

  def zero_row(i, carry):
    for cg in range(COLS // L):
      xbuf[i, pl.ds(cg * L, L)] = zero_v
    return carry
  lax.fori_loop(0, SEG_CH, zero_row, 0)

  for k in range(NSEG_CH):
    s0 = s * SEG_PT + k * SEG_CH
    pltpu.sync_copy(xbuf.at[pl.ds(0, SEG_CH)], sums_sh.at[pl.ds(s0, SEG_CH)])
    pltpu.sync_copy(xbuf.at[pl.ds(0, SEG_CH), pl.ds(0, L)],
                    counts_sh.at[pl.ds(s0, SEG_CH)])
  plsc.subcore_barrier()

  def chunk_a(i, carry):
    row0 = base + i * CH
    pltpu.sync_copy(x_hbm.at[pl.ds(row0, CH), pl.ds(col0, COLS)], xbuf)
    for j in range(CH // 128):
      pltpu.sync_copy(b_hbm.at[pl.ds(row0 + j * 128, 128)], idxbuf.at[j])
    for j in range(CH // 128):
      pltpu.sync_copy(xbuf.at[pl.ds(j * 128, 128)],
                      sums_sh.at[idxbuf.at[j]], add=True)
      pltpu.sync_copy(onesbuf, counts_sh.at[idxbuf.at[j]], add=True)
    return carry
  lax.fori_loop(0, NCH, chunk_a, 0)

  row_t = base + NCH * CH
  pltpu.sync_copy(x_hbm.at[pl.ds(row_t, TAIL), pl.ds(col0, COLS)],
                  xbuf.at[pl.ds(0, TAIL)])
  pltpu.sync_copy(b_hbm.at[pl.ds(row_t, TAIL)], tailidx)
  pltpu.sync_copy(xbuf.at[pl.ds(0, TAIL)], sums_sh.at[tailidx], add=True)
  pltpu.sync_copy(onesbuf.at[pl.ds(0, TAIL)], counts_sh.at[tailidx], add=True)
  plsc.subcore_barrier()

  for k in range(NSEG_CH):
    s0 = s * SEG_PT + k * SEG_CH
    pltpu.sync_copy(sums_sh.at[pl.ds(s0, SEG_CH)], xbuf.at[pl.ds(0, SEG_CH)])
    pltpu.sync_copy(counts_sh.at[pl.ds(s0, SEG_CH)],
                    obuf.at[pl.ds(0, SEG_CH), pl.ds(0, L)])

    def seg_row(r, carry):
      cnt = obuf[r, pl.ds(0, L)]
      scale = gamma_v / jnp.maximum(cnt, 1.0)
      for cg in range(COLS // L):
        gbuf[r, pl.ds(cg * L, L)] = xbuf[r, pl.ds(cg * L, L)] * scale
      return carry
    lax.fori_loop(0, SEG_CH, seg_row, 0)
    pltpu.sync_copy(gbuf.at[pl.ds(0, SEG_CH)], sums_sh.at[pl.ds(s0, SEG_CH)])
  plsc.subcore_barrier()

  def elemwise(nrows):
    def row_fn(r, carry):
      for cg in range(COLS // L):
        dsl = pl.ds(cg * L, L)
        obuf[r, dsl] = jnp.maximum(lamb_v * xbuf[r, dsl] + gbuf[r, dsl], 0.0)
      return carry
    lax.fori_loop(0, nrows, row_fn, 0)

  def chunk_c(i, carry):
    row0 = base + i * CH
    pltpu.sync_copy(x_hbm.at[pl.ds(row0, CH), pl.ds(col0, COLS)], xbuf)
    for j in range(CH // 128):
      pltpu.sync_copy(b_hbm.at[pl.ds(row0 + j * 128, 128)], idxbuf.at[j])
    for j in range(CH // 128):
      pltpu.sync_copy(sums_sh.at[idxbuf.at[j]], gbuf.at[pl.ds(j * 128, 128)])
    elemwise(CH)
    pltpu.sync_copy(obuf, out_hbm.at[pl.ds(row0, CH), pl.ds(col0, COLS)])
    return carry
  lax.fori_loop(0, NCH, chunk_c, 0)

  pltpu.sync_copy(x_hbm.at[pl.ds(row_t, TAIL), pl.ds(col0, COLS)],
                  xbuf.at[pl.ds(0, TAIL)])
  pltpu.sync_copy(b_hbm.at[pl.ds(row_t, TAIL)], tailidx)
  pltpu.sync_copy(sums_sh.at[tailidx], gbuf.at[pl.ds(0, TAIL)])
  elemwise(TAIL)
  pltpu.sync_copy(obuf.at[pl.ds(0, TAIL)],
                  out_hbm.at[pl.ds(row_t, TAIL), pl.ds(col0, COLS)])


@functools.partial(
    pl.kernel,
    out_type=jax.ShapeDtypeStruct((N, D), jnp.float32),
    mesh=plsc.VectorSubcoreMesh(core_axis_name="c", subcore_axis_name="s",
                                num_cores=NC, num_subcores=NS),
    scratch_types=[
        pltpu.VMEM_SHARED((S, COLS), jnp.float32),
        pltpu.VMEM_SHARED((S, L), jnp.float32),
        pltpu.VMEM((CH, COLS), jnp.float32),
        pltpu.VMEM((CH, COLS), jnp.float32),
        pltpu.VMEM((CH, COLS), jnp.float32),
        pltpu.VMEM((CH // 128, 128), jnp.int32),
        pltpu.VMEM((TAIL,), jnp.int32),
        pltpu.VMEM((128, L), jnp.float32),
        pltpu.VMEM((2 * L,), jnp.float32),
    ],
    compiler_params=pltpu.CompilerParams(use_tc_tiling_on_sc=False),
)
def _equivariant_sc(x_hbm, b_hbm, lg_hbm, out_hbm, *scratch):
  _body(x_hbm, b_hbm, lg_hbm, out_hbm, *scratch)


def kernel(x, batch, lamb, gamma):
  batch = batch.astype(jnp.int32)
  lg = jnp.concatenate([
      jnp.broadcast_to(lamb.astype(jnp.float32).reshape(()), (L,)),
      jnp.broadcast_to(gamma.astype(jnp.float32).reshape(()), (L,)),
  ])
  return _equivariant_sc(x, batch, lg)


# speedup vs baseline: 5.4518x; 1.6376x over previous
import functools

import jax
import jax.numpy as jnp
from jax import lax
from jax.experimental import pallas as pl
from jax.experimental.pallas import tpu as pltpu
from jax.experimental.pallas import tpu_sc as plsc

N = 320000
D = 128
S = 10000
NC = 2
NS = 16
L = 16

COLS = D // NC
BLK = 128
NBLK = N // BLK
BPT = NBLK // NS
CH = 256
NCH = 78
SEG_PT = S // NS
SEG_CH = 125
NSEG_CH = SEG_PT // SEG_CH


def _body(x_hbm, b2d_hbm, lg_hbm, out_hbm,
          sums_sh, counts_sh,
          xbuf, gbuf, idxbuf, onesbuf, cbuf, lgbuf,
          sx0, sx1, so0, so1):
  c = lax.axis_index("c")
  s = lax.axis_index("s")
  col0 = c * COLS
  base_blk = s * BPT + jnp.minimum(s, 4)
  sx = (sx0, sx1)
  so = (so0, so1)

  pltpu.sync_copy(lg_hbm, lgbuf)
  lamb_v = lgbuf[pl.ds(0, L)]
  gamma_v = lgbuf[pl.ds(L, L)]

  one_v = jnp.full((L,), 1.0, dtype=jnp.float32)
  zero_v = jnp.zeros((L,), dtype=jnp.float32)

  def init_row(i, carry):
    onesbuf[i, pl.ds(0, L)] = one_v
    for cg in range(COLS // L):
      xbuf[0, i, pl.ds(cg * L, L)] = zero_v
    return carry
  lax.fori_loop(0, SEG_CH, init_row, 0)

  def init_row2(i, carry):
    onesbuf[i, pl.ds(0, L)] = one_v
    return carry
  lax.fori_loop(SEG_CH, BLK, init_row2, 0)

  for k in range(NSEG_CH):
    s0 = s * SEG_PT + k * SEG_CH
    pltpu.sync_copy(xbuf.at[0, pl.ds(0, SEG_CH)], sums_sh.at[pl.ds(s0, SEG_CH)])
    pltpu.sync_copy(xbuf.at[0, pl.ds(0, SEG_CH), pl.ds(0, L)],
                    counts_sh.at[pl.ds(s0, SEG_CH)])
  plsc.subcore_barrier()

  def chunk_row0(i):
    return (base_blk + 2 * i) * BLK

  def start_x(i, b):
    pltpu.async_copy(x_hbm.at[pl.ds(chunk_row0(i), CH), pl.ds(col0, COLS)],
                     xbuf.at[b], sx[b])

  def wait_x(b):
    pltpu.make_async_copy(x_hbm.at[pl.ds(0, CH), pl.ds(col0, COLS)],
                          xbuf.at[b], sx[b]).wait()

  def load_idx(i, b):
    pltpu.sync_copy(b2d_hbm.at[pl.ds(base_blk + 2 * i, CH // BLK)],
                    idxbuf.at[b])

  def scatter_chunk(b):
    for j in range(CH // BLK):
      pltpu.sync_copy(xbuf.at[b, pl.ds(j * BLK, BLK)],
                      sums_sh.at[idxbuf.at[b, j]], add=True)
      pltpu.sync_copy(onesbuf, counts_sh.at[idxbuf.at[b, j]], add=True)

  start_x(0, 0)
  for b in range(2):
    i = b
    start_x(i + 1, 1 - b)
    load_idx(i, b)
    wait_x(b)
    scatter_chunk(b)

  def outer_a(k, carry):
    for b in range(2):
      i = 2 * k + 2 + b
      pl.when(i + 1 < NCH)(lambda: start_x(i + 1, 1 - b))
      load_idx(i, b)
      wait_x(b)
      scatter_chunk(b)
    return carry
  lax.fori_loop(0, (NCH - 2) // 2, outer_a, 0)

  @pl.when(s < 4)
  def _():
    row0 = (base_blk + 2 * NCH) * BLK
    pltpu.sync_copy(x_hbm.at[pl.ds(row0, BLK), pl.ds(col0, COLS)],
                    xbuf.at[0, pl.ds(0, BLK)])
    pltpu.sync_copy(b2d_hbm.at[pl.ds(base_blk + 2 * NCH, 1)],
                    idxbuf.at[0, pl.ds(0, 1)])
    pltpu.sync_copy(xbuf.at[0, pl.ds(0, BLK)],
                    sums_sh.at[idxbuf.at[0, 0]], add=True)
    pltpu.sync_copy(onesbuf, counts_sh.at[idxbuf.at[0, 0]], add=True)
  plsc.subcore_barrier()

  for k in range(NSEG_CH):
    s0 = s * SEG_PT + k * SEG_CH
    pltpu.sync_copy(sums_sh.at[pl.ds(s0, SEG_CH)], xbuf.at[0, pl.ds(0, SEG_CH)])
    pltpu.sync_copy(counts_sh.at[pl.ds(s0, SEG_CH)], cbuf)

    def seg_row(r, carry):
      cnt = cbuf[r, pl.ds(0, L)]
      scale = gamma_v / jnp.maximum(cnt, 1.0)
      for cg in range(COLS // L):
        gbuf[0, r, pl.ds(cg * L, L)] = xbuf[0, r, pl.ds(cg * L, L)] * scale
      return carry
    lax.fori_loop(0, SEG_CH, seg_row, 0)
    pltpu.sync_copy(gbuf.at[0, pl.ds(0, SEG_CH)], sums_sh.at[pl.ds(s0, SEG_CH)])
  plsc.subcore_barrier()

  def gather_chunk(b):
    for j in range(CH // BLK):
      pltpu.sync_copy(sums_sh.at[idxbuf.at[b, j]],
                      gbuf.at[b, pl.ds(j * BLK, BLK)])

  def elemwise(b, nrows):
    def row_fn(r, carry):
      for cg in range(COLS // L):
        dsl = pl.ds(cg * L, L)
        gbuf[b, r, dsl] = jnp.maximum(
            lamb_v * xbuf[b, r, dsl] + gbuf[b, r, dsl], 0.0)
      return carry
    lax.fori_loop(0, nrows, row_fn, 0)

  def start_store(i, b):
    pltpu.async_copy(gbuf.at[b],
                     out_hbm.at[pl.ds(chunk_row0(i), CH), pl.ds(col0, COLS)],
                     so[b])

  def wait_store(b):
    pltpu.make_async_copy(gbuf.at[b],
                          out_hbm.at[pl.ds(0, CH), pl.ds(col0, COLS)],
                          so[b]).wait()

  start_x(0, 0)
  for b in range(2):
    i = b
    start_x(i + 1, 1 - b)
    load_idx(i, b)
    wait_x(b)
    gather_chunk(b)
    elemwise(b, CH)
    start_store(i, b)

  def outer_c(k, carry):
    for b in range(2):
      i = 2 * k + 2 + b
      pl.when(i + 1 < NCH)(lambda: start_x(i + 1, 1 - b))
      load_idx(i, b)
      wait_x(b)
      wait_store(b)
      gather_chunk(b)
      elemwise(b, CH)
      start_store(i, b)
    return carry
  lax.fori_loop(0, (NCH - 2) // 2, outer_c, 0)
  wait_store(0)
  wait_store(1)

  @pl.when(s < 4)
  def _():
    row0 = (base_blk + 2 * NCH) * BLK
    pltpu.sync_copy(x_hbm.at[pl.ds(row0, BLK), pl.ds(col0, COLS)],
                    xbuf.at[0, pl.ds(0, BLK)])
    pltpu.sync_copy(b2d_hbm.at[pl.ds(base_blk + 2 * NCH, 1)],
                    idxbuf.at[0, pl.ds(0, 1)])
    pltpu.sync_copy(sums_sh.at[idxbuf.at[0, 0]], gbuf.at[0, pl.ds(0, BLK)])
    elemwise(0, BLK)
    pltpu.sync_copy(gbuf.at[0, pl.ds(0, BLK)],
                    out_hbm.at[pl.ds(row0, BLK), pl.ds(col0, COLS)])


@functools.partial(
    pl.kernel,
    out_type=jax.ShapeDtypeStruct((N, D), jnp.float32),
    mesh=plsc.VectorSubcoreMesh(core_axis_name="c", subcore_axis_name="s",
                                num_cores=NC, num_subcores=NS),
    scratch_types=[
        pltpu.VMEM_SHARED((S, COLS), jnp.float32),
        pltpu.VMEM_SHARED((S, L), jnp.float32),
        pltpu.VMEM((2, CH, COLS), jnp.float32),
        pltpu.VMEM((2, CH, COLS), jnp.float32),
        pltpu.VMEM((2, CH // BLK, BLK), jnp.int32),
        pltpu.VMEM((BLK, L), jnp.float32),
        pltpu.VMEM((SEG_CH, L), jnp.float32),
        pltpu.VMEM((2 * L,), jnp.float32),
        pltpu.SemaphoreType.DMA,
        pltpu.SemaphoreType.DMA,
        pltpu.SemaphoreType.DMA,
        pltpu.SemaphoreType.DMA,
    ],
    compiler_params=pltpu.CompilerParams(use_tc_tiling_on_sc=False),
)
def _equivariant_sc(x_hbm, b2d_hbm, lg_hbm, out_hbm, *scratch):
  _body(x_hbm, b2d_hbm, lg_hbm, out_hbm, *scratch)


def kernel(x, batch, lamb, gamma):
  batch = batch.astype(jnp.int32)
  b2d = batch.reshape(NBLK, BLK)
  lg = jnp.concatenate([
      jnp.broadcast_to(lamb.astype(jnp.float32).reshape(()), (L,)),
      jnp.broadcast_to(gamma.astype(jnp.float32).reshape(()), (L,)),
  ])
  return _equivariant_sc(x, b2d, lg)


# speedup vs baseline: 6.8203x; 1.2510x over previous
import functools

import jax
import jax.numpy as jnp
from jax import lax
from jax.experimental import pallas as pl
from jax.experimental.pallas import tpu as pltpu
from jax.experimental.pallas import tpu_sc as plsc

N = 320000
D = 128
S = 10000
NC = 2
NS = 16
L = 16

COLS = D // NC
BLK = 128
NBLK = N // BLK
BPT = NBLK // NS
CH = 256
NCH = 78
SEG_PT = S // NS
SEG_CH = 125
NSEG_CH = SEG_PT // SEG_CH


def _body(x_hbm, b2d_hbm, lg_hbm, out_hbm,
          sums_sh, counts_sh,
          xbuf, gbuf, idxbuf, onesbuf, cbuf, lgbuf,
          sx0, sx1, si0, si1, so0, so1, sg0, sg1, sc0, sc1):
  c = lax.axis_index("c")
  s = lax.axis_index("s")
  col0 = c * COLS
  base_blk = s * BPT + jnp.minimum(s, 4)
  sx = (sx0, sx1)
  si = (si0, si1)
  so = (so0, so1)
  sg = (sg0, sg1)
  sc = (sc0, sc1)

  pltpu.sync_copy(lg_hbm, lgbuf)
  lamb_v = lgbuf[pl.ds(0, L)]
  gamma_v = lgbuf[pl.ds(L, L)]

  one_v = jnp.full((L,), 1.0, dtype=jnp.float32)
  zero_v = jnp.zeros((L,), dtype=jnp.float32)

  def init_row(i, carry):
    onesbuf[i, pl.ds(0, L)] = one_v
    for cg in range(COLS // L):
      xbuf[0, i, pl.ds(cg * L, L)] = zero_v
    return carry
  lax.fori_loop(0, SEG_CH, init_row, 0)

  def init_row2(i, carry):
    onesbuf[i, pl.ds(0, L)] = one_v
    return carry
  lax.fori_loop(SEG_CH, BLK, init_row2, 0)

  for k in range(NSEG_CH):
    s0 = s * SEG_PT + k * SEG_CH
    pltpu.sync_copy(xbuf.at[0, pl.ds(0, SEG_CH)], sums_sh.at[pl.ds(s0, SEG_CH)])
    pltpu.sync_copy(xbuf.at[0, pl.ds(0, SEG_CH), pl.ds(0, L)],
                    counts_sh.at[pl.ds(s0, SEG_CH)])
  plsc.subcore_barrier()

  def chunk_row0(i):
    return (base_blk + 2 * i) * BLK

  def x_copy(i, b):
    return pltpu.make_async_copy(
        x_hbm.at[pl.ds(chunk_row0(i), CH), pl.ds(col0, COLS)],
        xbuf.at[b], sx[b])

  def idx_copy(i, b):
    return pltpu.make_async_copy(
        b2d_hbm.at[pl.ds(base_blk + 2 * i, CH // BLK)], idxbuf.at[b], si[b])

  def start_scatters(b):
    for j in range(CH // BLK):
      pltpu.async_copy(xbuf.at[b, pl.ds(j * BLK, BLK)],
                       sums_sh.at[idxbuf.at[b, j]], sc[b], add=True)
      pltpu.async_copy(onesbuf, counts_sh.at[idxbuf.at[b, j]], sc[b], add=True)

  def wait_scatters(b):
    for j in range(CH // BLK):
      pltpu.make_async_copy(xbuf.at[b, pl.ds(j * BLK, BLK)],
                            sums_sh.at[idxbuf.at[b, j]], sc[b]).wait()
      pltpu.make_async_copy(onesbuf, counts_sh.at[idxbuf.at[b, j]],
                            sc[b]).wait()

  def start_gathers(b):
    for j in range(CH // BLK):
      pltpu.async_copy(sums_sh.at[idxbuf.at[b, j]],
                       gbuf.at[b, pl.ds(j * BLK, BLK)], sg[b])

  def wait_gathers(b):
    for j in range(CH // BLK):
      pltpu.make_async_copy(sums_sh.at[idxbuf.at[b, j]],
                            gbuf.at[b, pl.ds(j * BLK, BLK)], sg[b]).wait()

  def store_copy(i, b):
    return pltpu.make_async_copy(
        gbuf.at[b],
        out_hbm.at[pl.ds(chunk_row0(i), CH), pl.ds(col0, COLS)], so[b])

  x_copy(0, 0).start()
  idx_copy(0, 0).start()

  def body_a(i, b):
    @pl.when(i >= 1)
    def _():
      wait_scatters(1 - b)
    @pl.when(i + 1 < NCH)
    def _():
      x_copy(i + 1, 1 - b).start()
      idx_copy(i + 1, 1 - b).start()
    x_copy(i, b).wait()
    idx_copy(i, b).wait()
    start_scatters(b)

  def outer_a(k, carry):
    for b in range(2):
      body_a(2 * k + b, b)
    return carry
  lax.fori_loop(0, NCH // 2, outer_a, 0)
  wait_scatters(1)

  @pl.when(s < 4)
  def _():
    row0 = (base_blk + 2 * NCH) * BLK
    pltpu.sync_copy(x_hbm.at[pl.ds(row0, BLK), pl.ds(col0, COLS)],
                    xbuf.at[0, pl.ds(0, BLK)])
    pltpu.sync_copy(b2d_hbm.at[pl.ds(base_blk + 2 * NCH, 1)],
                    idxbuf.at[0, pl.ds(0, 1)])
    pltpu.sync_copy(xbuf.at[0, pl.ds(0, BLK)],
                    sums_sh.at[idxbuf.at[0, 0]], add=True)
    pltpu.sync_copy(onesbuf, counts_sh.at[idxbuf.at[0, 0]], add=True)
  plsc.subcore_barrier()

  for k in range(NSEG_CH):
    s0 = s * SEG_PT + k * SEG_CH
    pltpu.sync_copy(sums_sh.at[pl.ds(s0, SEG_CH)], xbuf.at[0, pl.ds(0, SEG_CH)])
    pltpu.sync_copy(counts_sh.at[pl.ds(s0, SEG_CH)], cbuf)

    def seg_row(r, carry):
      cnt = cbuf[r, pl.ds(0, L)]
      scale = gamma_v / jnp.maximum(cnt, 1.0)
      for cg in range(COLS // L):
        gbuf[0, r, pl.ds(cg * L, L)] = xbuf[0, r, pl.ds(cg * L, L)] * scale
      return carry
    lax.fori_loop(0, SEG_CH, seg_row, 0)
    pltpu.sync_copy(gbuf.at[0, pl.ds(0, SEG_CH)], sums_sh.at[pl.ds(s0, SEG_CH)])
  plsc.subcore_barrier()

  def elemwise(b, nrows):
    def row_fn(r, carry):
      for cg in range(COLS // L):
        dsl = pl.ds(cg * L, L)
        gbuf[b, r, dsl] = jnp.maximum(
            lamb_v * xbuf[b, r, dsl] + gbuf[b, r, dsl], 0.0)
      return carry
    lax.fori_loop(0, nrows, row_fn, 0)

  x_copy(0, 0).start()
  idx_copy(0, 0).start()
  idx_copy(0, 0).wait()
  start_gathers(0)

  def body_c(i, b):
    @pl.when(i + 1 < NCH)
    def _():
      x_copy(i + 1, 1 - b).start()
      idx_copy(i + 1, 1 - b).start()
    x_copy(i, b).wait()
    wait_gathers(b)
    elemwise(b, CH)
    store_copy(i, b).start()
    @pl.when(i + 1 < NCH)
    def _():
      idx_copy(i + 1, 1 - b).wait()
      @pl.when(i >= 1)
      def _():
        store_copy(0, 1 - b).wait()
      start_gathers(1 - b)

  def outer_c(k, carry):
    for b in range(2):
      body_c(2 * k + b, b)
    return carry
  lax.fori_loop(0, NCH // 2, outer_c, 0)
  store_copy(0, 0).wait()
  store_copy(0, 1).wait()

  @pl.when(s < 4)
  def _():
    row0 = (base_blk + 2 * NCH) * BLK
    pltpu.sync_copy(x_hbm.at[pl.ds(row0, BLK), pl.ds(col0, COLS)],
                    xbuf.at[0, pl.ds(0, BLK)])
    pltpu.sync_copy(b2d_hbm.at[pl.ds(base_blk + 2 * NCH, 1)],
                    idxbuf.at[0, pl.ds(0, 1)])
    pltpu.sync_copy(sums_sh.at[idxbuf.at[0, 0]], gbuf.at[0, pl.ds(0, BLK)])
    elemwise(0, BLK)
    pltpu.sync_copy(gbuf.at[0, pl.ds(0, BLK)],
                    out_hbm.at[pl.ds(row0, BLK), pl.ds(col0, COLS)])


@functools.partial(
    pl.kernel,
    out_type=jax.ShapeDtypeStruct((N, D), jnp.float32),
    mesh=plsc.VectorSubcoreMesh(core_axis_name="c", subcore_axis_name="s",
                                num_cores=NC, num_subcores=NS),
    scratch_types=[
        pltpu.VMEM_SHARED((S, COLS), jnp.float32),
        pltpu.VMEM_SHARED((S, L), jnp.float32),
        pltpu.VMEM((2, CH, COLS), jnp.float32),
        pltpu.VMEM((2, CH, COLS), jnp.float32),
        pltpu.VMEM((2, CH // BLK, BLK), jnp.int32),
        pltpu.VMEM((BLK, L), jnp.float32),
        pltpu.VMEM((SEG_CH, L), jnp.float32),
        pltpu.VMEM((2 * L,), jnp.float32),
        pltpu.SemaphoreType.DMA,
        pltpu.SemaphoreType.DMA,
        pltpu.SemaphoreType.DMA,
        pltpu.SemaphoreType.DMA,
        pltpu.SemaphoreType.DMA,
        pltpu.SemaphoreType.DMA,
        pltpu.SemaphoreType.DMA,
        pltpu.SemaphoreType.DMA,
        pltpu.SemaphoreType.DMA,
        pltpu.SemaphoreType.DMA,
    ],
    compiler_params=pltpu.CompilerParams(use_tc_tiling_on_sc=False),
)
def _equivariant_sc(x_hbm, b2d_hbm, lg_hbm, out_hbm, *scratch):
  _body(x_hbm, b2d_hbm, lg_hbm, out_hbm, *scratch)


def kernel(x, batch, lamb, gamma):
  batch = batch.astype(jnp.int32)
  b2d = batch.reshape(NBLK, BLK)
  lg = jnp.concatenate([
      jnp.broadcast_to(lamb.astype(jnp.float32).reshape(()), (L,)),
      jnp.broadcast_to(gamma.astype(jnp.float32).reshape(()), (L,)),
  ])
  return _equivariant_sc(x, b2d, lg)


# speedup vs baseline: 7.0553x; 1.0345x over previous
import functools

import jax
import jax.numpy as jnp
from jax import lax
from jax.experimental import pallas as pl
from jax.experimental.pallas import tpu as pltpu
from jax.experimental.pallas import tpu_sc as plsc

N = 320000
D = 128
S = 10000
NC = 2
NS = 16
L = 16

COLS = D // NC
BLK = 128
NBLK = N // BLK
BPT = NBLK // NS
CH = 256
NCH = 78
SEG_PT = S // NS
SEG_CH = 125
NSEG_CH = SEG_PT // SEG_CH


def _body(x_hbm, b2d_hbm, lg_hbm, out_hbm,
          sums_sh, counts_sh,
          xbuf, gbuf, idxbuf, onesbuf, cbuf, lgbuf,
          sx0, sx1, si0, si1, so0, so1, sg0, sg1, sc0, sc1):
  c = lax.axis_index("c")
  s = lax.axis_index("s")
  col0 = c * COLS
  base_blk = s * BPT + jnp.minimum(s, 4)
  sx = (sx0, sx1)
  si = (si0, si1)
  so = (so0, so1)
  sg = (sg0, sg1)
  sc = (sc0, sc1)

  pltpu.sync_copy(lg_hbm, lgbuf)
  lamb_v = lgbuf[pl.ds(0, L)]
  gamma_v = lgbuf[pl.ds(L, L)]

  one_v = jnp.full((L,), 1.0, dtype=jnp.float32)
  zero_v = jnp.zeros((L,), dtype=jnp.float32)

  def init_row(i, carry):
    onesbuf[i, pl.ds(0, L)] = one_v
    for cg in range(COLS // L):
      xbuf[0, i, pl.ds(cg * L, L)] = zero_v
    return carry
  lax.fori_loop(0, SEG_CH, init_row, 0)

  def init_row2(i, carry):
    onesbuf[i, pl.ds(0, L)] = one_v
    return carry
  lax.fori_loop(SEG_CH, BLK, init_row2, 0)

  for k in range(NSEG_CH):
    s0 = s * SEG_PT + k * SEG_CH
    pltpu.sync_copy(xbuf.at[0, pl.ds(0, SEG_CH)], sums_sh.at[pl.ds(s0, SEG_CH)])
    pltpu.sync_copy(xbuf.at[0, pl.ds(0, SEG_CH), pl.ds(0, L)],
                    counts_sh.at[pl.ds(s0, SEG_CH)])
  plsc.subcore_barrier()

  def chunk_row0(i):
    return (base_blk + 2 * i) * BLK

  def x_copy(i, b):
    return pltpu.make_async_copy(
        x_hbm.at[pl.ds(chunk_row0(i), CH), pl.ds(col0, COLS)],
        xbuf.at[b], sx[b])

  def idx_copy(i, b):
    return pltpu.make_async_copy(
        b2d_hbm.at[pl.ds(base_blk + 2 * i, CH // BLK)], idxbuf.at[b], si[b])

  def start_scatters(b):
    for j in range(CH // BLK):
      pltpu.async_copy(xbuf.at[b, pl.ds(j * BLK, BLK)],
                       sums_sh.at[idxbuf.at[b, j]], sc[b], add=True)
      pltpu.async_copy(onesbuf, counts_sh.at[idxbuf.at[b, j]], sc[b], add=True)

  def wait_scatters(b):
    for j in range(CH // BLK):
      pltpu.make_async_copy(xbuf.at[b, pl.ds(j * BLK, BLK)],
                            sums_sh.at[idxbuf.at[b, j]], sc[b]).wait()
      pltpu.make_async_copy(onesbuf, counts_sh.at[idxbuf.at[b, j]],
                            sc[b]).wait()

  def start_gathers(b):
    for j in range(CH // BLK):
      pltpu.async_copy(sums_sh.at[idxbuf.at[b, j]],
                       gbuf.at[b, pl.ds(j * BLK, BLK)], sg[b])

  def wait_gathers(b):
    for j in range(CH // BLK):
      pltpu.make_async_copy(sums_sh.at[idxbuf.at[b, j]],
                            gbuf.at[b, pl.ds(j * BLK, BLK)], sg[b]).wait()

  def store_copy(i, b):
    return pltpu.make_async_copy(
        gbuf.at[b],
        out_hbm.at[pl.ds(chunk_row0(i), CH), pl.ds(col0, COLS)], so[b])

  x_copy(0, 0).start()
  idx_copy(0, 0).start()

  def body_a(i, b):
    @pl.when(i >= 1)
    def _():
      wait_scatters(1 - b)
    @pl.when(i + 1 < NCH)
    def _():
      x_copy(i + 1, 1 - b).start()
      idx_copy(i + 1, 1 - b).start()
    x_copy(i, b).wait()
    idx_copy(i, b).wait()
    start_scatters(b)

  def outer_a(k, carry):
    for b in range(2):
      body_a(2 * k + b, b)
    return carry
  lax.fori_loop(0, NCH // 2, outer_a, 0)
  wait_scatters(1)

  @pl.when(s < 4)
  def _():
    row0 = (base_blk + 2 * NCH) * BLK
    pltpu.sync_copy(x_hbm.at[pl.ds(row0, BLK), pl.ds(col0, COLS)],
                    xbuf.at[0, pl.ds(0, BLK)])
    pltpu.sync_copy(b2d_hbm.at[pl.ds(base_blk + 2 * NCH, 1)],
                    idxbuf.at[0, pl.ds(0, 1)])
    pltpu.sync_copy(xbuf.at[0, pl.ds(0, BLK)],
                    sums_sh.at[idxbuf.at[0, 0]], add=True)
    pltpu.sync_copy(onesbuf, counts_sh.at[idxbuf.at[0, 0]], add=True)
  plsc.subcore_barrier()

  for k in range(NSEG_CH):
    s0 = s * SEG_PT + k * SEG_CH
    pltpu.sync_copy(sums_sh.at[pl.ds(s0, SEG_CH)], xbuf.at[0, pl.ds(0, SEG_CH)])
    pltpu.sync_copy(counts_sh.at[pl.ds(s0, SEG_CH)], cbuf)

    def seg_row(r, carry):
      cnt = cbuf[r, pl.ds(0, L)]
      scale = gamma_v / jnp.maximum(cnt, 1.0)
      for cg in range(COLS // L):
        gbuf[0, r, pl.ds(cg * L, L)] = xbuf[0, r, pl.ds(cg * L, L)] * scale
      return carry
    lax.fori_loop(0, SEG_CH, seg_row, 0)
    pltpu.sync_copy(gbuf.at[0, pl.ds(0, SEG_CH)], sums_sh.at[pl.ds(s0, SEG_CH)])
  plsc.subcore_barrier()

  def elemwise(b, nrows):
    @plsc.parallel_loop(0, nrows, step=1, unroll=2)
    def row_fn(r):
      for cg in range(COLS // L):
        dsl = pl.ds(cg * L, L)
        gbuf[b, r, dsl] = jnp.maximum(
            lamb_v * xbuf[b, r, dsl] + gbuf[b, r, dsl], 0.0)

  x_copy(0, 0).start()
  idx_copy(0, 0).start()
  idx_copy(0, 0).wait()
  start_gathers(0)

  def body_c(i, b):
    @pl.when(i + 1 < NCH)
    def _():
      x_copy(i + 1, 1 - b).start()
      idx_copy(i + 1, 1 - b).start()
    x_copy(i, b).wait()
    wait_gathers(b)
    elemwise(b, CH)
    store_copy(i, b).start()
    @pl.when(i + 1 < NCH)
    def _():
      idx_copy(i + 1, 1 - b).wait()
      @pl.when(i >= 1)
      def _():
        store_copy(0, 1 - b).wait()
      start_gathers(1 - b)

  def outer_c(k, carry):
    for b in range(2):
      body_c(2 * k + b, b)
    return carry
  lax.fori_loop(0, NCH // 2, outer_c, 0)
  store_copy(0, 0).wait()
  store_copy(0, 1).wait()

  @pl.when(s < 4)
  def _():
    row0 = (base_blk + 2 * NCH) * BLK
    pltpu.sync_copy(x_hbm.at[pl.ds(row0, BLK), pl.ds(col0, COLS)],
                    xbuf.at[0, pl.ds(0, BLK)])
    pltpu.sync_copy(b2d_hbm.at[pl.ds(base_blk + 2 * NCH, 1)],
                    idxbuf.at[0, pl.ds(0, 1)])
    pltpu.sync_copy(sums_sh.at[idxbuf.at[0, 0]], gbuf.at[0, pl.ds(0, BLK)])
    elemwise(0, BLK)
    pltpu.sync_copy(gbuf.at[0, pl.ds(0, BLK)],
                    out_hbm.at[pl.ds(row0, BLK), pl.ds(col0, COLS)])


@functools.partial(
    pl.kernel,
    out_type=jax.ShapeDtypeStruct((N, D), jnp.float32),
    mesh=plsc.VectorSubcoreMesh(core_axis_name="c", subcore_axis_name="s",
                                num_cores=NC, num_subcores=NS),
    scratch_types=[
        pltpu.VMEM_SHARED((S, COLS), jnp.float32),
        pltpu.VMEM_SHARED((S, L), jnp.float32),
        pltpu.VMEM((2, CH, COLS), jnp.float32),
        pltpu.VMEM((2, CH, COLS), jnp.float32),
        pltpu.VMEM((2, CH // BLK, BLK), jnp.int32),
        pltpu.VMEM((BLK, L), jnp.float32),
        pltpu.VMEM((SEG_CH, L), jnp.float32),
        pltpu.VMEM((2 * L,), jnp.float32),
        pltpu.SemaphoreType.DMA,
        pltpu.SemaphoreType.DMA,
        pltpu.SemaphoreType.DMA,
        pltpu.SemaphoreType.DMA,
        pltpu.SemaphoreType.DMA,
        pltpu.SemaphoreType.DMA,
        pltpu.SemaphoreType.DMA,
        pltpu.SemaphoreType.DMA,
        pltpu.SemaphoreType.DMA,
        pltpu.SemaphoreType.DMA,
    ],
    compiler_params=pltpu.CompilerParams(use_tc_tiling_on_sc=False),
)
def _equivariant_sc(x_hbm, b2d_hbm, lg_hbm, out_hbm, *scratch):
  _body(x_hbm, b2d_hbm, lg_hbm, out_hbm, *scratch)


def kernel(x, batch, lamb, gamma):
  batch = batch.astype(jnp.int32)
  b2d = batch.reshape(NBLK, BLK)
  lg = jnp.concatenate([
      jnp.broadcast_to(lamb.astype(jnp.float32).reshape(()), (L,)),
      jnp.broadcast_to(gamma.astype(jnp.float32).reshape(()), (L,)),
  ])
  return _equivariant_sc(x, b2d, lg)


# speedup vs baseline: 7.4314x; 1.0533x over previous
import functools

import jax
import jax.numpy as jnp
from jax import lax
from jax.experimental import pallas as pl
from jax.experimental.pallas import tpu as pltpu
from jax.experimental.pallas import tpu_sc as plsc

N = 320000
D = 128
S = 10000
NC = 2
NS = 16
L = 16

COLS = D // NC
BLK = 128
NBLK = N // BLK
BPT = NBLK // NS
CH = 256
NCH = 78
SEG_PT = S // NS
SEG_CH = 125
NSEG_CH = SEG_PT // SEG_CH
SH = 640


def _body(x_hbm, b2d_hbm, lg_hbm, out_hbm,
          sums_sh, cnt2_sh,
          xbuf, gbuf, idxbuf, hbuf, idrows, cbuf, lgbuf,
          sx0, sx1, si0, si1, so0, so1, sg0, sg1, sc0, sc1):
  c = lax.axis_index("c")
  s = lax.axis_index("s")
  col0 = c * COLS
  base_blk = s * BPT + jnp.minimum(s, 4)
  sx = (sx0, sx1)
  si = (si0, si1)
  so = (so0, so1)
  sg = (sg0, sg1)
  sc = (sc0, sc1)

  pltpu.sync_copy(lg_hbm, lgbuf)
  lamb_v = lgbuf[pl.ds(0, L)]
  gamma_v = lgbuf[pl.ds(L, L)]

  one_v = jnp.full((L,), 1.0, dtype=jnp.float32)
  zero_v = jnp.zeros((L,), dtype=jnp.float32)
  iota_v = lax.iota(jnp.int32, L)

  def init_row(i, carry):
    for cg in range(COLS // L):
      xbuf[0, i, pl.ds(cg * L, L)] = zero_v
    return carry
  lax.fori_loop(0, SEG_CH, init_row, 0)

  def init_hist(i, carry):
    hbuf[i, pl.ds(0, L)] = zero_v
    return carry
  lax.fori_loop(0, SH, init_hist, 0)

  for j in range(SH // BLK):
    for g in range(BLK // L):
      idrows[j, pl.ds(g * L, L)] = j * BLK + g * L + iota_v

  for k in range(NSEG_CH):
    s0 = s * SEG_PT + k * SEG_CH
    pltpu.sync_copy(xbuf.at[0, pl.ds(0, SEG_CH)], sums_sh.at[pl.ds(s0, SEG_CH)])
  pltpu.sync_copy(xbuf.at[0, pl.ds(0, SH // NS), pl.ds(0, L)],
                  cnt2_sh.at[pl.ds(s * (SH // NS), SH // NS)])
  plsc.subcore_barrier()

  def chunk_row0(i):
    return (base_blk + 2 * i) * BLK

  def x_copy(i, b):
    return pltpu.make_async_copy(
        x_hbm.at[pl.ds(chunk_row0(i), CH), pl.ds(col0, COLS)],
        xbuf.at[b], sx[b])

  def idx_copy(i, b):
    return pltpu.make_async_copy(
        b2d_hbm.at[pl.ds(base_blk + 2 * i, CH // BLK)], idxbuf.at[b], si[b])

  def start_scatters(b):
    for j in range(CH // BLK):
      pltpu.async_copy(xbuf.at[b, pl.ds(j * BLK, BLK)],
                       sums_sh.at[idxbuf.at[b, j]], sc[b], add=True)

  def wait_scatters(b):
    for j in range(CH // BLK):
      pltpu.make_async_copy(xbuf.at[b, pl.ds(j * BLK, BLK)],
                            sums_sh.at[idxbuf.at[b, j]], sc[b]).wait()

  def hist_accumulate(b):
    for j in range(CH // BLK):
      for g in range(BLK // L):
        iv = idxbuf[b, j, pl.ds(g * L, L)]
        plsc.addupdate_scatter(
            hbuf, [lax.shift_right_logical(iv, 4), iv & 15], one_v)

  def start_gathers(b):
    for j in range(CH // BLK):
      pltpu.async_copy(sums_sh.at[idxbuf.at[b, j]],
                       gbuf.at[b, pl.ds(j * BLK, BLK)], sg[b])

  def wait_gathers(b):
    for j in range(CH // BLK):
      pltpu.make_async_copy(sums_sh.at[idxbuf.at[b, j]],
                            gbuf.at[b, pl.ds(j * BLK, BLK)], sg[b]).wait()

  def store_copy(i, b):
    return pltpu.make_async_copy(
        gbuf.at[b],
        out_hbm.at[pl.ds(chunk_row0(i), CH), pl.ds(col0, COLS)], so[b])

  x_copy(0, 0).start()
  idx_copy(0, 0).start()

  def body_a(i, b):
    @pl.when(i >= 1)
    def _():
      wait_scatters(1 - b)
    @pl.when(i + 1 < NCH)
    def _():
      x_copy(i + 1, 1 - b).start()
      idx_copy(i + 1, 1 - b).start()
    x_copy(i, b).wait()
    idx_copy(i, b).wait()
    start_scatters(b)
    hist_accumulate(b)

  def outer_a(k, carry):
    for b in range(2):
      body_a(2 * k + b, b)
    return carry
  lax.fori_loop(0, NCH // 2, outer_a, 0)
  wait_scatters(1)

  @pl.when(s < 4)
  def _():
    row0 = (base_blk + 2 * NCH) * BLK
    pltpu.sync_copy(x_hbm.at[pl.ds(row0, BLK), pl.ds(col0, COLS)],
                    xbuf.at[0, pl.ds(0, BLK)])
    pltpu.sync_copy(b2d_hbm.at[pl.ds(base_blk + 2 * NCH, 1)],
                    idxbuf.at[0, pl.ds(0, 1)])
    pltpu.sync_copy(xbuf.at[0, pl.ds(0, BLK)],
                    sums_sh.at[idxbuf.at[0, 0]], add=True)
    for g in range(BLK // L):
      iv = idxbuf[0, 0, pl.ds(g * L, L)]
      plsc.addupdate_scatter(
          hbuf, [lax.shift_right_logical(iv, 4), iv & 15], one_v)

  for j in range(SH // BLK):
    pltpu.sync_copy(hbuf.at[pl.ds(j * BLK, BLK)],
                    cnt2_sh.at[idrows.at[j]], add=True)
  plsc.subcore_barrier()

  r0 = (SEG_PT * s) // L
  pltpu.sync_copy(cnt2_sh.at[pl.ds(r0, SEG_PT // L + 1)], cbuf)
  for k in range(NSEG_CH):
    s0 = s * SEG_PT + k * SEG_CH
    pltpu.sync_copy(sums_sh.at[pl.ds(s0, SEG_CH)], xbuf.at[0, pl.ds(0, SEG_CH)])

    def seg_row(r, carry):
      flat = s + k * SEG_CH + r
      cnt_row = cbuf[lax.shift_right_logical(flat, 4), pl.ds(0, L)]
      lane = jnp.full((L,), flat & 15, dtype=jnp.int32)
      cnt = cnt_row.at[lane].get(mode="promise_in_bounds")
      scale = gamma_v / jnp.maximum(cnt, 1.0)
      for cg in range(COLS // L):
        gbuf[0, r, pl.ds(cg * L, L)] = xbuf[0, r, pl.ds(cg * L, L)] * scale
      return carry
    lax.fori_loop(0, SEG_CH, seg_row, 0)
    pltpu.sync_copy(gbuf.at[0, pl.ds(0, SEG_CH)], sums_sh.at[pl.ds(s0, SEG_CH)])
  plsc.subcore_barrier()

  def elemwise(b, nrows):
    @plsc.parallel_loop(0, nrows, step=1, unroll=2)
    def row_fn(r):
      for cg in range(COLS // L):
        dsl = pl.ds(cg * L, L)
        gbuf[b, r, dsl] = jnp.maximum(
            lamb_v * xbuf[b, r, dsl] + gbuf[b, r, dsl], 0.0)

  x_copy(0, 0).start()
  idx_copy(0, 0).start()
  idx_copy(0, 0).wait()
  start_gathers(0)

  def body_c(i, b):
    @pl.when(i + 1 < NCH)
    def _():
      x_copy(i + 1, 1 - b).start()
      idx_copy(i + 1, 1 - b).start()
    x_copy(i, b).wait()
    wait_gathers(b)
    elemwise(b, CH)
    store_copy(i, b).start()
    @pl.when(i + 1 < NCH)
    def _():
      idx_copy(i + 1, 1 - b).wait()
      @pl.when(i >= 1)
      def _():
        store_copy(0, 1 - b).wait()
      start_gathers(1 - b)

  def outer_c(k, carry):
    for b in range(2):
      body_c(2 * k + b, b)
    return carry
  lax.fori_loop(0, NCH // 2, outer_c, 0)
  store_copy(0, 0).wait()
  store_copy(0, 1).wait()

  @pl.when(s < 4)
  def _():
    row0 = (base_blk + 2 * NCH) * BLK
    pltpu.sync_copy(x_hbm.at[pl.ds(row0, BLK), pl.ds(col0, COLS)],
                    xbuf.at[0, pl.ds(0, BLK)])
    pltpu.sync_copy(b2d_hbm.at[pl.ds(base_blk + 2 * NCH, 1)],
                    idxbuf.at[0, pl.ds(0, 1)])
    pltpu.sync_copy(sums_sh.at[idxbuf.at[0, 0]], gbuf.at[0, pl.ds(0, BLK)])
    elemwise(0, BLK)
    pltpu.sync_copy(gbuf.at[0, pl.ds(0, BLK)],
                    out_hbm.at[pl.ds(row0, BLK), pl.ds(col0, COLS)])


@functools.partial(
    pl.kernel,
    out_type=jax.ShapeDtypeStruct((N, D), jnp.float32),
    mesh=plsc.VectorSubcoreMesh(core_axis_name="c", subcore_axis_name="s",
                                num_cores=NC, num_subcores=NS),
    scratch_types=[
        pltpu.VMEM_SHARED((S, COLS), jnp.float32),
        pltpu.VMEM_SHARED((SH, L), jnp.float32),
        pltpu.VMEM((2, CH, COLS), jnp.float32),
        pltpu.VMEM((2, CH, COLS), jnp.float32),
        pltpu.VMEM((2, CH // BLK, BLK), jnp.int32),
        pltpu.VMEM((SH, L), jnp.float32),
        pltpu.VMEM((SH // BLK, BLK), jnp.int32),
        pltpu.VMEM((SEG_PT // L + 1, L), jnp.float32),
        pltpu.VMEM((2 * L,), jnp.float32),
        pltpu.SemaphoreType.DMA,
        pltpu.SemaphoreType.DMA,
        pltpu.SemaphoreType.DMA,
        pltpu.SemaphoreType.DMA,
        pltpu.SemaphoreType.DMA,
        pltpu.SemaphoreType.DMA,
        pltpu.SemaphoreType.DMA,
        pltpu.SemaphoreType.DMA,
        pltpu.SemaphoreType.DMA,
        pltpu.SemaphoreType.DMA,
    ],
    compiler_params=pltpu.CompilerParams(use_tc_tiling_on_sc=False,
                                         needs_layout_passes=False),
)
def _equivariant_sc(x_hbm, b2d_hbm, lg_hbm, out_hbm, *scratch):
  _body(x_hbm, b2d_hbm, lg_hbm, out_hbm, *scratch)


def kernel(x, batch, lamb, gamma):
  batch = batch.astype(jnp.int32)
  b2d = batch.reshape(NBLK, BLK)
  lg = jnp.concatenate([
      jnp.broadcast_to(lamb.astype(jnp.float32).reshape(()), (L,)),
      jnp.broadcast_to(gamma.astype(jnp.float32).reshape(()), (L,)),
  ])
  return _equivariant_sc(x, b2d, lg)
